# baseline (speedup 1.0000x reference)
import functools

import jax
import jax.numpy as jnp
from jax import lax
from jax.experimental import pallas as pl
from jax.experimental.pallas import tpu as pltpu
from jax.experimental.pallas import tpu_sc as plsc

N_NODES = 10000
N_EDGES = 320000
D = 128
NC = 2
NS = 16
NW = NC * NS
SUB = 128
EPADW = 10240
EPAD = NW * EPADW
QPT = EPADW // SUB
IDXROWS = EPAD // SUB
NPAD = 10240
NRCHUNK = N_NODES // 8
RITERS = -(-NRCHUNK // NS)

_mesh = plsc.VectorSubcoreMesh(
    core_axis_name="c", subcore_axis_name="s", num_cores=NC, num_subcores=NS)


@functools.partial(
    pl.kernel,
    out_type=jax.ShapeDtypeStruct((NC, 2, NPAD), jnp.float32),
    mesh=_mesh,
    scratch_types=[
        pltpu.VMEM((QPT, SUB), jnp.int32),
        pltpu.VMEM((QPT, SUB), jnp.int32),
        pltpu.VMEM((SUB,), jnp.float32),
        pltpu.VMEM((640,), jnp.float32),
        pltpu.VMEM_SHARED((NPAD,), jnp.float32),
        pltpu.VMEM_SHARED((NPAD,), jnp.float32),
        pltpu.SemaphoreType.DMA,
        pltpu.SemaphoreType.DMA,
    ],
)
def _degree_kernel(src_hbm, dst_hbm, out_hbm, srcv, dstv, ones_v, zeros_v,
                   hist_s, hist_d, isem, ssem):
    c = lax.axis_index("c")
    s = lax.axis_index("s")
    w = c * NS + s

    pltpu.async_copy(src_hbm.at[pl.ds(w * QPT, QPT)], srcv, isem)
    pltpu.async_copy(dst_hbm.at[pl.ds(w * QPT, QPT)], dstv, isem)

    def fill_zeros(i, _):
        zeros_v[pl.ds(i * 16, 16)] = jnp.zeros((16,), jnp.float32)
        return 0
    lax.fori_loop(0, 640 // 16, fill_zeros, 0)

    def fill_ones(i, _):
        ones_v[pl.ds(i * 16, 16)] = jnp.ones((16,), jnp.float32)
        return 0
    lax.fori_loop(0, SUB // 16, fill_ones, 0)

    pltpu.sync_copy(zeros_v, hist_s.at[pl.ds(s * 640, 640)])
    pltpu.sync_copy(zeros_v, hist_d.at[pl.ds(s * 640, 640)])
    pltpu.make_async_copy(src_hbm.at[pl.ds(w * QPT, QPT)], srcv, isem).wait()
    pltpu.make_async_copy(dst_hbm.at[pl.ds(w * QPT, QPT)], dstv, isem).wait()
    plsc.subcore_barrier()

    def batch(t, _):
        def fire(q_, _2):
            q = t * 8 + q_
            pltpu.async_copy(ones_v, hist_s.at[srcv.at[q]], ssem, add=True)
            pltpu.async_copy(ones_v, hist_d.at[dstv.at[q]], ssem, add=True)
            return 0
        lax.fori_loop(0, 8, fire, 0)

        def drain(q_, _2):
            pltpu.make_async_copy(ones_v, hist_s.at[srcv.at[0]], ssem).wait()
            pltpu.make_async_copy(ones_v, hist_d.at[dstv.at[0]], ssem).wait()
            return 0
        lax.fori_loop(0, 8, drain, 0)
        return 0
    lax.fori_loop(0, QPT // 8, batch, 0)
    plsc.subcore_barrier()

    @pl.when(s == 0)
    def _():
        pltpu.sync_copy(hist_s, out_hbm.at[c, 0])

    @pl.when(s == 1)
    def _():
        pltpu.sync_copy(hist_d, out_hbm.at[c, 1])


BROWS = 8
NBLK = QPT // BROWS


@functools.partial(
    pl.kernel,
    out_type=jax.ShapeDtypeStruct((NC, N_NODES, D), jnp.float32),
    mesh=_mesh,
    scratch_types=[
        pltpu.VMEM((2, BROWS, SUB), jnp.int32),
        pltpu.VMEM((2, BROWS, SUB), jnp.int32),
        pltpu.VMEM((2, SUB, D), jnp.float32),
        pltpu.VMEM((8, D), jnp.float32),
        pltpu.VMEM_SHARED((N_NODES + 8, D), jnp.float32),
        pltpu.SemaphoreType.DMA,
        pltpu.SemaphoreType.DMA,
        pltpu.SemaphoreType.DMA,
    ],
)
def _agg_kernel(y_hbm, src_hbm, dst_hbm, out_hbm, srcb, dstb, rows, zrows,
                agg, isem0, isem1, gsem):
    c = lax.axis_index("c")
    s = lax.axis_index("s")
    w = c * NS + s
    isems = (isem0, isem1)

    def fire_idx(t, p):
        base = w * QPT + t * BROWS
        pltpu.async_copy(src_hbm.at[pl.ds(base, BROWS)], srcb.at[p], isems[p])
        pltpu.async_copy(dst_hbm.at[pl.ds(base, BROWS)], dstb.at[p], isems[p])

    def wait_idx(p):
        pltpu.make_async_copy(src_hbm.at[pl.ds(0, BROWS)], srcb.at[p],
                              isems[p]).wait()
        pltpu.make_async_copy(dst_hbm.at[pl.ds(0, BROWS)], dstb.at[p],
                              isems[p]).wait()

    fire_idx(0, 0)
    fire_idx(1, 1)

    def fill_zrows(i, _):
        zrows[i // 8, pl.ds((i % 8) * 16, 16)] = jnp.zeros((16,), jnp.float32)
        return 0
    lax.fori_loop(0, 8 * 8, fill_zrows, 0)

    def zero_agg(i, _):
        j = i * NS + s

        @pl.when(j < NRCHUNK)
        def _():
            pltpu.sync_copy(zrows, agg.at[pl.ds(j * 8, 8)])
        return 0
    lax.fori_loop(0, RITERS, zero_agg, 0)
    plsc.subcore_barrier()

    wait_idx(0)
    pltpu.async_copy(y_hbm.at[srcb.at[0, 0]], rows.at[0], gsem)

    def sblock(i, _):
        for p in range(2):
            t = i * 2 + p
            for q in range(BROWS):
                bb = q % 2
                pltpu.make_async_copy(y_hbm.at[srcb.at[p, 0]], rows.at[bb],
                                      gsem).wait()
                if q < BROWS - 1:
                    pltpu.async_copy(y_hbm.at[srcb.at[p, q + 1]],
                                     rows.at[1 - bb], gsem)
                else:
                    @pl.when(t < NBLK - 1)
                    def _():
                        wait_idx(1 - p)
                        pltpu.async_copy(y_hbm.at[srcb.at[1 - p, 0]],
                                         rows.at[1 - bb], gsem)
                pltpu.sync_copy(rows.at[bb], agg.at[dstb.at[p, q]], add=True)

            @pl.when(t + 2 < NBLK)
            def _():
                fire_idx(t + 2, p)
        return 0
    lax.fori_loop(0, NBLK // 2, sblock, 0)
    plsc.subcore_barrier()

    def copy_out(i, _):
        j = i * NS + s

        @pl.when(j < NRCHUNK)
        def _():
            pltpu.sync_copy(agg.at[pl.ds(j * 8, 8)],
                            out_hbm.at[c, pl.ds(j * 8, 8)])
        return 0
    lax.fori_loop(0, RITERS, copy_out, 0)


_RB = 2048


def _prescale_matmul_body(deg_ref, x_ref, w_ref, y_ref):
    d = deg_ref[0, 0, :] + deg_ref[1, 0, :]
    ns = lax.rsqrt(jnp.maximum(d, 1.0))
    y_ref[...] = jnp.dot(x_ref[...] * ns[:, None], w_ref[...],
                         preferred_element_type=jnp.float32)


def _finish_body(deg_ref, b_ref, p_ref, o_ref):
    d = deg_ref[0, 1, :] + deg_ref[1, 1, :]
    nd = lax.rsqrt(jnp.maximum(d, 1.0))
    o_ref[...] = (p_ref[0] + p_ref[1]) * nd[:, None] + b_ref[...]


def kernel(features, edge_index, W, b):
    edge_index = edge_index.astype(jnp.int32)
    pad = jnp.full((EPAD - N_EDGES,), N_NODES, jnp.int32)
    src = jnp.concatenate([edge_index[0], pad]).reshape(IDXROWS, SUB)
    dst = jnp.concatenate([edge_index[1], pad]).reshape(IDXROWS, SUB)

    deg = _degree_kernel(src, dst)

    y = pl.pallas_call(
        _prescale_matmul_body,
        grid=(NPAD // _RB,),
        in_specs=[
            pl.BlockSpec((NC, 2, _RB), lambda i: (0, 0, i)),
            pl.BlockSpec((_RB, D), lambda i: (i, 0)),
            pl.BlockSpec((D, D), lambda i: (0, 0)),
        ],
        out_specs=pl.BlockSpec((_RB, D), lambda i: (i, 0)),
        out_shape=jax.ShapeDtypeStruct((NPAD, D), jnp.float32),
    )(deg, features, W)

    parts = _agg_kernel(y, src, dst)

    out = pl.pallas_call(
        _finish_body,
        grid=(pl.cdiv(N_NODES, _RB),),
        in_specs=[
            pl.BlockSpec((NC, 2, _RB), lambda i: (0, 0, i)),
            pl.BlockSpec((1, D), lambda i: (0, 0)),
            pl.BlockSpec((NC, _RB, D), lambda i: (0, i, 0)),
        ],
        out_specs=pl.BlockSpec((_RB, D), lambda i: (i, 0)),
        out_shape=jax.ShapeDtypeStruct((N_NODES, D), jnp.float32),
    )(deg, b.reshape(1, D), parts)

    return out


# speedup vs baseline: 2.8379x; 2.8379x over previous
import functools

import jax
import jax.numpy as jnp
from jax import lax
from jax.experimental import pallas as pl
from jax.experimental.pallas import tpu as pltpu
from jax.experimental.pallas import tpu_sc as plsc

N_NODES = 10000
N_EDGES = 320000
D = 128
NC = 2
NS = 16
NW = NC * NS
SUB = 128
EPADW = 10240
EPAD = NW * EPADW
QPT = EPADW // SUB
IDXROWS = EPAD // SUB
REALROWS = N_EDGES // SUB
NPAD = 10240
NRCHUNK = N_NODES // 8
RITERS = -(-NRCHUNK // NS)

_mesh = plsc.VectorSubcoreMesh(
    core_axis_name="c", subcore_axis_name="s", num_cores=NC, num_subcores=NS)


@functools.partial(
    pl.kernel,
    out_type=jax.ShapeDtypeStruct((NC, 2, NPAD), jnp.float32),
    mesh=_mesh,
    scratch_types=[
        pltpu.VMEM((QPT, SUB), jnp.int32),
        pltpu.VMEM((QPT, SUB), jnp.int32),
        pltpu.VMEM((SUB,), jnp.float32),
        pltpu.VMEM((640,), jnp.float32),
        pltpu.VMEM_SHARED((NPAD,), jnp.float32),
        pltpu.VMEM_SHARED((NPAD,), jnp.float32),
        pltpu.SemaphoreType.DMA,
        pltpu.SemaphoreType.DMA,
    ],
)
def _degree_kernel(src_hbm, dst_hbm, out_hbm, srcv, dstv, ones_v, zeros_v,
                   hist_s, hist_d, isem, ssem):
    c = lax.axis_index("c")
    s = lax.axis_index("s")
    w = c * NS + s

    pltpu.async_copy(src_hbm.at[pl.ds(w * QPT, QPT)], srcv, isem)
    pltpu.async_copy(dst_hbm.at[pl.ds(w * QPT, QPT)], dstv, isem)

    def fill_zeros(i, _):
        zeros_v[pl.ds(i * 16, 16)] = jnp.zeros((16,), jnp.float32)
        return 0
    lax.fori_loop(0, 640 // 16, fill_zeros, 0)

    def fill_ones(i, _):
        ones_v[pl.ds(i * 16, 16)] = jnp.ones((16,), jnp.float32)
        return 0
    lax.fori_loop(0, SUB // 16, fill_ones, 0)

    pltpu.sync_copy(zeros_v, hist_s.at[pl.ds(s * 640, 640)])
    pltpu.sync_copy(zeros_v, hist_d.at[pl.ds(s * 640, 640)])
    pltpu.make_async_copy(src_hbm.at[pl.ds(w * QPT, QPT)], srcv, isem).wait()
    pltpu.make_async_copy(dst_hbm.at[pl.ds(w * QPT, QPT)], dstv, isem).wait()
    plsc.subcore_barrier()

    def batch(t, _):
        def fire(q_, _2):
            q = t * 8 + q_

            @pl.when(w * QPT + q < REALROWS)
            def _():
                pltpu.async_copy(ones_v, hist_s.at[srcv.at[q]], ssem,
                                 add=True)
                pltpu.async_copy(ones_v, hist_d.at[dstv.at[q]], ssem,
                                 add=True)
            return 0
        lax.fori_loop(0, 8, fire, 0)

        def drain(q_, _2):
            q = t * 8 + q_

            @pl.when(w * QPT + q < REALROWS)
            def _():
                pltpu.make_async_copy(ones_v, hist_s.at[srcv.at[0]],
                                      ssem).wait()
                pltpu.make_async_copy(ones_v, hist_d.at[dstv.at[0]],
                                      ssem).wait()
            return 0
        lax.fori_loop(0, 8, drain, 0)
        return 0
    lax.fori_loop(0, QPT // 8, batch, 0)
    plsc.subcore_barrier()

    @pl.when(s == 0)
    def _():
        pltpu.sync_copy(hist_s, out_hbm.at[c, 0])

    @pl.when(s == 1)
    def _():
        pltpu.sync_copy(hist_d, out_hbm.at[c, 1])


BROWS = 8
NBLK = QPT // BROWS


@functools.partial(
    pl.kernel,
    out_type=jax.ShapeDtypeStruct((NC, N_NODES, D), jnp.float32),
    mesh=_mesh,
    scratch_types=[
        pltpu.VMEM((2, BROWS, SUB), jnp.int32),
        pltpu.VMEM((2, BROWS, SUB), jnp.int32),
        pltpu.VMEM((2, SUB, D), jnp.float32),
        pltpu.VMEM((8, D), jnp.float32),
        pltpu.VMEM_SHARED((N_NODES, D), jnp.float32),
        pltpu.SemaphoreType.DMA,
        pltpu.SemaphoreType.DMA,
        pltpu.SemaphoreType.DMA,
    ],
)
def _agg_kernel(y_hbm, src_hbm, dst_hbm, out_hbm, srcb, dstb, rows, zrows,
                agg, isem0, isem1, gsem):
    c = lax.axis_index("c")
    s = lax.axis_index("s")
    w = c * NS + s
    isems = (isem0, isem1)

    def fire_idx(t, p):
        base = w * QPT + t * BROWS
        pltpu.async_copy(src_hbm.at[pl.ds(base, BROWS)], srcb.at[p], isems[p])
        pltpu.async_copy(dst_hbm.at[pl.ds(base, BROWS)], dstb.at[p], isems[p])

    def wait_idx(p):
        pltpu.make_async_copy(src_hbm.at[pl.ds(0, BROWS)], srcb.at[p],
                              isems[p]).wait()
        pltpu.make_async_copy(dst_hbm.at[pl.ds(0, BROWS)], dstb.at[p],
                              isems[p]).wait()

    fire_idx(0, 0)
    fire_idx(1, 1)

    def fill_zrows(i, _):
        zrows[i // 8, pl.ds((i % 8) * 16, 16)] = jnp.zeros((16,), jnp.float32)
        return 0
    lax.fori_loop(0, 8 * 8, fill_zrows, 0)

    def zero_agg(i, _):
        j = i * NS + s

        @pl.when(j < NRCHUNK)
        def _():
            pltpu.sync_copy(zrows, agg.at[pl.ds(j * 8, 8)])
        return 0
    lax.fori_loop(0, RITERS, zero_agg, 0)
    plsc.subcore_barrier()

    wait_idx(0)
    pltpu.async_copy(y_hbm.at[srcb.at[0, 0]], rows.at[0], gsem)

    def sblock(i, _):
        for p in range(2):
            t = i * 2 + p
            for q in range(BROWS):
                bb = q % 2
                rr = w * QPT + t * BROWS + q

                @pl.when(rr < REALROWS)
                def _():
                    pltpu.make_async_copy(y_hbm.at[srcb.at[p, 0]],
                                          rows.at[bb], gsem).wait()
                if q < BROWS - 1:
                    @pl.when(rr + 1 < REALROWS)
                    def _():
                        pltpu.async_copy(y_hbm.at[srcb.at[p, q + 1]],
                                         rows.at[1 - bb], gsem)
                else:
                    @pl.when(t < NBLK - 1)
                    def _():
                        wait_idx(1 - p)

                    @pl.when((t < NBLK - 1) & (rr + 1 < REALROWS))
                    def _():
                        pltpu.async_copy(y_hbm.at[srcb.at[1 - p, 0]],
                                         rows.at[1 - bb], gsem)

                @pl.when(rr < REALROWS)
                def _():
                    pltpu.sync_copy(rows.at[bb], agg.at[dstb.at[p, q]],
                                    add=True)

            @pl.when(t + 2 < NBLK)
            def _():
                fire_idx(t + 2, p)
        return 0
    lax.fori_loop(0, NBLK // 2, sblock, 0)
    plsc.subcore_barrier()

    def copy_out(i, _):
        j = i * NS + s

        @pl.when(j < NRCHUNK)
        def _():
            pltpu.sync_copy(agg.at[pl.ds(j * 8, 8)],
                            out_hbm.at[c, pl.ds(j * 8, 8)])
        return 0
    lax.fori_loop(0, RITERS, copy_out, 0)


_RB = 2048


def _prescale_matmul_body(deg_ref, x_ref, w_ref, y_ref):
    d = deg_ref[0, 0, :] + deg_ref[1, 0, :]
    ns = lax.rsqrt(jnp.maximum(d, 1.0))
    y_ref[...] = jnp.dot(x_ref[...] * ns[:, None], w_ref[...],
                         preferred_element_type=jnp.float32)


def _finish_body(deg_ref, b_ref, p_ref, o_ref):
    d = deg_ref[0, 1, :] + deg_ref[1, 1, :]
    nd = lax.rsqrt(jnp.maximum(d, 1.0))
    o_ref[...] = (p_ref[0] + p_ref[1]) * nd[:, None] + b_ref[...]


def kernel(features, edge_index, W, b):
    edge_index = edge_index.astype(jnp.int32)
    pad = jnp.full((EPAD - N_EDGES,), N_NODES, jnp.int32)
    src = jnp.concatenate([edge_index[0], pad]).reshape(IDXROWS, SUB)
    dst = jnp.concatenate([edge_index[1], pad]).reshape(IDXROWS, SUB)

    deg = _degree_kernel(src, dst)

    y = pl.pallas_call(
        _prescale_matmul_body,
        grid=(NPAD // _RB,),
        in_specs=[
            pl.BlockSpec((NC, 2, _RB), lambda i: (0, 0, i)),
            pl.BlockSpec((_RB, D), lambda i: (i, 0)),
            pl.BlockSpec((D, D), lambda i: (0, 0)),
        ],
        out_specs=pl.BlockSpec((_RB, D), lambda i: (i, 0)),
        out_shape=jax.ShapeDtypeStruct((NPAD, D), jnp.float32),
    )(deg, features, W)

    parts = _agg_kernel(y, src, dst)

    out = pl.pallas_call(
        _finish_body,
        grid=(pl.cdiv(N_NODES, _RB),),
        in_specs=[
            pl.BlockSpec((NC, 2, _RB), lambda i: (0, 0, i)),
            pl.BlockSpec((1, D), lambda i: (0, 0)),
            pl.BlockSpec((NC, _RB, D), lambda i: (0, i, 0)),
        ],
        out_specs=pl.BlockSpec((_RB, D), lambda i: (i, 0)),
        out_shape=jax.ShapeDtypeStruct((N_NODES, D), jnp.float32),
    )(deg, b.reshape(1, D), parts)

    return out


# speedup vs baseline: 3.5243x; 1.2419x over previous
import functools

import jax
import jax.numpy as jnp
from jax import lax
from jax.experimental import pallas as pl
from jax.experimental.pallas import tpu as pltpu
from jax.experimental.pallas import tpu_sc as plsc

N_NODES = 10000
N_EDGES = 320000
D = 128
NC = 2
NS = 16
NW = NC * NS
SUB = 128
EPADW = 10240
EPAD = NW * EPADW
QPT = EPADW // SUB
IDXROWS = EPAD // SUB
REALROWS = N_EDGES // SUB
NPAD = 10240
NRCHUNK = N_NODES // 8
RITERS = -(-NRCHUNK // NS)

_mesh = plsc.VectorSubcoreMesh(
    core_axis_name="c", subcore_axis_name="s", num_cores=NC, num_subcores=NS)


@functools.partial(
    pl.kernel,
    out_type=jax.ShapeDtypeStruct((NC, 2, NPAD), jnp.float32),
    mesh=_mesh,
    scratch_types=[
        pltpu.VMEM((QPT, SUB), jnp.int32),
        pltpu.VMEM((QPT, SUB), jnp.int32),
        pltpu.VMEM((SUB,), jnp.float32),
        pltpu.VMEM((640,), jnp.float32),
        pltpu.VMEM_SHARED((NPAD,), jnp.float32),
        pltpu.VMEM_SHARED((NPAD,), jnp.float32),
        pltpu.SemaphoreType.DMA,
        pltpu.SemaphoreType.DMA,
    ],
)
def _degree_kernel(src_hbm, dst_hbm, out_hbm, srcv, dstv, ones_v, zeros_v,
                   hist_s, hist_d, isem, ssem):
    c = lax.axis_index("c")
    s = lax.axis_index("s")
    w = c * NS + s

    pltpu.async_copy(src_hbm.at[pl.ds(w * QPT, QPT)], srcv, isem)
    pltpu.async_copy(dst_hbm.at[pl.ds(w * QPT, QPT)], dstv, isem)

    def fill_zeros(i, _):
        zeros_v[pl.ds(i * 16, 16)] = jnp.zeros((16,), jnp.float32)
        return 0
    lax.fori_loop(0, 640 // 16, fill_zeros, 0)

    def fill_ones(i, _):
        ones_v[pl.ds(i * 16, 16)] = jnp.ones((16,), jnp.float32)
        return 0
    lax.fori_loop(0, SUB // 16, fill_ones, 0)

    pltpu.sync_copy(zeros_v, hist_s.at[pl.ds(s * 640, 640)])
    pltpu.sync_copy(zeros_v, hist_d.at[pl.ds(s * 640, 640)])
    pltpu.make_async_copy(src_hbm.at[pl.ds(w * QPT, QPT)], srcv, isem).wait()
    pltpu.make_async_copy(dst_hbm.at[pl.ds(w * QPT, QPT)], dstv, isem).wait()
    plsc.subcore_barrier()

    def batch(t, _):
        def fire(q_, _2):
            q = t * 8 + q_

            @pl.when(w * QPT + q < REALROWS)
            def _():
                pltpu.async_copy(ones_v, hist_s.at[srcv.at[q]], ssem,
                                 add=True)
                pltpu.async_copy(ones_v, hist_d.at[dstv.at[q]], ssem,
                                 add=True)
            return 0
        lax.fori_loop(0, 8, fire, 0)

        def drain(q_, _2):
            q = t * 8 + q_

            @pl.when(w * QPT + q < REALROWS)
            def _():
                pltpu.make_async_copy(ones_v, hist_s.at[srcv.at[0]],
                                      ssem).wait()
                pltpu.make_async_copy(ones_v, hist_d.at[dstv.at[0]],
                                      ssem).wait()
            return 0
        lax.fori_loop(0, 8, drain, 0)
        return 0
    lax.fori_loop(0, QPT // 8, batch, 0)
    plsc.subcore_barrier()

    @pl.when(s == 0)
    def _():
        pltpu.sync_copy(hist_s, out_hbm.at[c, 0])

    @pl.when(s == 1)
    def _():
        pltpu.sync_copy(hist_d, out_hbm.at[c, 1])


BROWS = 8
NBLK = QPT // BROWS


@functools.partial(
    pl.kernel,
    out_type=jax.ShapeDtypeStruct((NC, N_NODES, D), jnp.float32),
    mesh=_mesh,
    scratch_types=[
        pltpu.VMEM((2, BROWS, SUB), jnp.int32),
        pltpu.VMEM((2, BROWS, SUB), jnp.int32),
        pltpu.VMEM((2, SUB, D), jnp.float32),
        pltpu.VMEM((8, D), jnp.float32),
        pltpu.VMEM_SHARED((N_NODES, D), jnp.float32),
        pltpu.SemaphoreType.DMA,
        pltpu.SemaphoreType.DMA,
        pltpu.SemaphoreType.DMA,
        pltpu.SemaphoreType.DMA,
    ],
)
def _agg_kernel(y_hbm, src_hbm, dst_hbm, out_hbm, srcb, dstb, rows, zrows,
                agg, isem0, isem1, gsem, ssem):
    c = lax.axis_index("c")
    s = lax.axis_index("s")
    w = c * NS + s
    isems = (isem0, isem1)

    def fire_idx(t, p):
        base = w * QPT + t * BROWS
        pltpu.async_copy(src_hbm.at[pl.ds(base, BROWS)], srcb.at[p], isems[p])
        pltpu.async_copy(dst_hbm.at[pl.ds(base, BROWS)], dstb.at[p], isems[p])

    def wait_idx(p):
        pltpu.make_async_copy(src_hbm.at[pl.ds(0, BROWS)], srcb.at[p],
                              isems[p]).wait()
        pltpu.make_async_copy(dst_hbm.at[pl.ds(0, BROWS)], dstb.at[p],
                              isems[p]).wait()

    fire_idx(0, 0)
    fire_idx(1, 1)

    def fill_zrows(i, _):
        zrows[i // 8, pl.ds((i % 8) * 16, 16)] = jnp.zeros((16,), jnp.float32)
        return 0
    lax.fori_loop(0, 8 * 8, fill_zrows, 0)

    def zero_batch(t, _):
        def fire(i_, _2):
            j = (t * 8 + i_) * NS + s

            @pl.when(j < NRCHUNK)
            def _():
                pltpu.async_copy(zrows, agg.at[pl.ds(j * 8, 8)], ssem)
            return 0
        lax.fori_loop(0, 8, fire, 0)

        def drain(i_, _2):
            j = (t * 8 + i_) * NS + s

            @pl.when(j < NRCHUNK)
            def _():
                pltpu.make_async_copy(zrows, agg.at[pl.ds(0, 8)],
                                      ssem).wait()
            return 0
        lax.fori_loop(0, 8, drain, 0)
        return 0
    lax.fori_loop(0, -(-RITERS // 8), zero_batch, 0)
    plsc.subcore_barrier()

    wait_idx(0)
    pltpu.async_copy(y_hbm.at[srcb.at[0, 0]], rows.at[0], gsem)

    def wait_scat(bb):
        pltpu.make_async_copy(rows.at[bb], agg.at[dstb.at[0, 0]],
                              ssem).wait()

    def sblock(i, _):
        for p in range(2):
            t = i * 2 + p
            for q in range(BROWS):
                bb = q % 2
                rr = w * QPT + t * BROWS + q

                @pl.when(rr < REALROWS)
                def _():
                    pltpu.make_async_copy(y_hbm.at[srcb.at[p, 0]],
                                          rows.at[bb], gsem).wait()
                if q == 0:
                    @pl.when((t > 0) & (rr < REALROWS))
                    def _():
                        wait_scat(1 - bb)
                else:
                    @pl.when(rr < REALROWS)
                    def _():
                        wait_scat(1 - bb)

                @pl.when(rr < REALROWS)
                def _():
                    pltpu.async_copy(rows.at[bb], agg.at[dstb.at[p, q]],
                                     ssem, add=True)
                if q < BROWS - 1:
                    @pl.when(rr + 1 < REALROWS)
                    def _():
                        pltpu.async_copy(y_hbm.at[srcb.at[p, q + 1]],
                                         rows.at[1 - bb], gsem)
                else:
                    @pl.when(t < NBLK - 1)
                    def _():
                        wait_idx(1 - p)

                    @pl.when((t < NBLK - 1) & (rr + 1 < REALROWS))
                    def _():
                        pltpu.async_copy(y_hbm.at[srcb.at[1 - p, 0]],
                                         rows.at[1 - bb], gsem)

            @pl.when(t + 2 < NBLK)
            def _():
                fire_idx(t + 2, p)
        return 0
    lax.fori_loop(0, NBLK // 2, sblock, 0)
    wait_scat(0)
    plsc.subcore_barrier()

    def out_batch(t, _):
        def fire(i_, _2):
            j = (t * 8 + i_) * NS + s

            @pl.when(j < NRCHUNK)
            def _():
                pltpu.async_copy(agg.at[pl.ds(j * 8, 8)],
                                 out_hbm.at[c, pl.ds(j * 8, 8)], gsem)
            return 0
        lax.fori_loop(0, 8, fire, 0)

        def drain(i_, _2):
            j = (t * 8 + i_) * NS + s

            @pl.when(j < NRCHUNK)
            def _():
                pltpu.make_async_copy(agg.at[pl.ds(0, 8)],
                                      out_hbm.at[c, pl.ds(0, 8)],
                                      gsem).wait()
            return 0
        lax.fori_loop(0, 8, drain, 0)
        return 0
    lax.fori_loop(0, -(-RITERS // 8), out_batch, 0)


_RB = 2048


def _prescale_matmul_body(deg_ref, x_ref, w_ref, y_ref):
    d = deg_ref[0, 0, :] + deg_ref[1, 0, :]
    ns = lax.rsqrt(jnp.maximum(d, 1.0))
    y_ref[...] = jnp.dot(x_ref[...] * ns[:, None], w_ref[...],
                         preferred_element_type=jnp.float32)


def _finish_body(deg_ref, b_ref, p_ref, o_ref):
    d = deg_ref[0, 1, :] + deg_ref[1, 1, :]
    nd = lax.rsqrt(jnp.maximum(d, 1.0))
    o_ref[...] = (p_ref[0] + p_ref[1]) * nd[:, None] + b_ref[...]


def kernel(features, edge_index, W, b):
    edge_index = edge_index.astype(jnp.int32)
    pad = jnp.full((EPAD - N_EDGES,), N_NODES, jnp.int32)
    src = jnp.concatenate([edge_index[0], pad]).reshape(IDXROWS, SUB)
    dst = jnp.concatenate([edge_index[1], pad]).reshape(IDXROWS, SUB)

    deg = _degree_kernel(src, dst)

    y = pl.pallas_call(
        _prescale_matmul_body,
        grid=(NPAD // _RB,),
        in_specs=[
            pl.BlockSpec((NC, 2, _RB), lambda i: (0, 0, i)),
            pl.BlockSpec((_RB, D), lambda i: (i, 0)),
            pl.BlockSpec((D, D), lambda i: (0, 0)),
        ],
        out_specs=pl.BlockSpec((_RB, D), lambda i: (i, 0)),
        out_shape=jax.ShapeDtypeStruct((NPAD, D), jnp.float32),
    )(deg, features, W)

    parts = _agg_kernel(y, src, dst)

    out = pl.pallas_call(
        _finish_body,
        grid=(pl.cdiv(N_NODES, _RB),),
        in_specs=[
            pl.BlockSpec((NC, 2, _RB), lambda i: (0, 0, i)),
            pl.BlockSpec((1, D), lambda i: (0, 0)),
            pl.BlockSpec((NC, _RB, D), lambda i: (0, i, 0)),
        ],
        out_specs=pl.BlockSpec((_RB, D), lambda i: (i, 0)),
        out_shape=jax.ShapeDtypeStruct((N_NODES, D), jnp.float32),
    )(deg, b.reshape(1, D), parts)

    return out


# speedup vs baseline: 3.5275x; 1.0009x over previous
import functools

import jax
import jax.numpy as jnp
from jax import lax
from jax.experimental import pallas as pl
from jax.experimental.pallas import tpu as pltpu
from jax.experimental.pallas import tpu_sc as plsc

N_NODES = 10000
N_EDGES = 320000
D = 128
NC = 2
NS = 16
NW = NC * NS
SUB = 128
EPADW = 10240
EPAD = NW * EPADW
QPT = EPADW // SUB
IDXROWS = EPAD // SUB
REALROWS = N_EDGES // SUB
NPAD = 10240
NRCHUNK = N_NODES // 8
RITERS = -(-NRCHUNK // NS)

_mesh = plsc.VectorSubcoreMesh(
    core_axis_name="c", subcore_axis_name="s", num_cores=NC, num_subcores=NS)


@functools.partial(
    pl.kernel,
    out_type=jax.ShapeDtypeStruct((NC, 2, NPAD), jnp.float32),
    mesh=_mesh,
    scratch_types=[
        pltpu.VMEM((QPT, SUB), jnp.int32),
        pltpu.VMEM((QPT, SUB), jnp.int32),
        pltpu.VMEM((SUB,), jnp.float32),
        pltpu.VMEM((640,), jnp.float32),
        pltpu.VMEM_SHARED((NPAD,), jnp.float32),
        pltpu.VMEM_SHARED((NPAD,), jnp.float32),
        pltpu.SemaphoreType.DMA,
        pltpu.SemaphoreType.DMA,
    ],
)
def _degree_kernel(src_hbm, dst_hbm, out_hbm, srcv, dstv, ones_v, zeros_v,
                   hist_s, hist_d, isem, ssem):
    c = lax.axis_index("c")
    s = lax.axis_index("s")
    w = c * NS + s

    pltpu.async_copy(src_hbm.at[pl.ds(w * QPT, QPT)], srcv, isem)
    pltpu.async_copy(dst_hbm.at[pl.ds(w * QPT, QPT)], dstv, isem)

    def fill_zeros(i, _):
        zeros_v[pl.ds(i * 16, 16)] = jnp.zeros((16,), jnp.float32)
        return 0
    lax.fori_loop(0, 640 // 16, fill_zeros, 0)

    def fill_ones(i, _):
        ones_v[pl.ds(i * 16, 16)] = jnp.ones((16,), jnp.float32)
        return 0
    lax.fori_loop(0, SUB // 16, fill_ones, 0)

    pltpu.sync_copy(zeros_v, hist_s.at[pl.ds(s * 640, 640)])
    pltpu.sync_copy(zeros_v, hist_d.at[pl.ds(s * 640, 640)])
    pltpu.make_async_copy(src_hbm.at[pl.ds(w * QPT, QPT)], srcv, isem).wait()
    pltpu.make_async_copy(dst_hbm.at[pl.ds(w * QPT, QPT)], dstv, isem).wait()
    plsc.subcore_barrier()

    def batch(t, _):
        def fire(q_, _2):
            q = t * 8 + q_

            @pl.when(w * QPT + q < REALROWS)
            def _():
                pltpu.async_copy(ones_v, hist_s.at[srcv.at[q]], ssem,
                                 add=True)
                pltpu.async_copy(ones_v, hist_d.at[dstv.at[q]], ssem,
                                 add=True)
            return 0
        lax.fori_loop(0, 8, fire, 0)

        def drain(q_, _2):
            q = t * 8 + q_

            @pl.when(w * QPT + q < REALROWS)
            def _():
                pltpu.make_async_copy(ones_v, hist_s.at[srcv.at[0]],
                                      ssem).wait()
                pltpu.make_async_copy(ones_v, hist_d.at[dstv.at[0]],
                                      ssem).wait()
            return 0
        lax.fori_loop(0, 8, drain, 0)
        return 0
    lax.fori_loop(0, QPT // 8, batch, 0)
    plsc.subcore_barrier()

    pltpu.sync_copy(hist_s.at[pl.ds(s * 640, 640)],
                    out_hbm.at[c, 0, pl.ds(s * 640, 640)])
    pltpu.sync_copy(hist_d.at[pl.ds(s * 640, 640)],
                    out_hbm.at[c, 1, pl.ds(s * 640, 640)])


BROWS = 8
NBLK = QPT // BROWS


@functools.partial(
    pl.kernel,
    out_type=jax.ShapeDtypeStruct((NC, N_NODES, D), jnp.float32),
    mesh=_mesh,
    scratch_types=[
        pltpu.VMEM((2, BROWS, SUB), jnp.int32),
        pltpu.VMEM((2, BROWS, SUB), jnp.int32),
        pltpu.VMEM((2, SUB, D), jnp.float32),
        pltpu.VMEM((8, D), jnp.float32),
        pltpu.VMEM_SHARED((N_NODES, D), jnp.float32),
        pltpu.SemaphoreType.DMA,
        pltpu.SemaphoreType.DMA,
        pltpu.SemaphoreType.DMA,
        pltpu.SemaphoreType.DMA,
    ],
)
def _agg_kernel(y_hbm, src_hbm, dst_hbm, out_hbm, srcb, dstb, rows, zrows,
                agg, isem0, isem1, gsem, ssem):
    c = lax.axis_index("c")
    s = lax.axis_index("s")
    w = c * NS + s
    isems = (isem0, isem1)

    def fire_idx(t, p):
        base = w * QPT + t * BROWS
        pltpu.async_copy(src_hbm.at[pl.ds(base, BROWS)], srcb.at[p], isems[p])
        pltpu.async_copy(dst_hbm.at[pl.ds(base, BROWS)], dstb.at[p], isems[p])

    def wait_idx(p):
        pltpu.make_async_copy(src_hbm.at[pl.ds(0, BROWS)], srcb.at[p],
                              isems[p]).wait()
        pltpu.make_async_copy(dst_hbm.at[pl.ds(0, BROWS)], dstb.at[p],
                              isems[p]).wait()

    fire_idx(0, 0)
    fire_idx(1, 1)

    def fill_zrows(i, _):
        zrows[i // 8, pl.ds((i % 8) * 16, 16)] = jnp.zeros((16,), jnp.float32)
        return 0
    lax.fori_loop(0, 8 * 8, fill_zrows, 0)

    def zero_batch(t, _):
        def fire(i_, _2):
            j = (t * 8 + i_) * NS + s

            @pl.when(j < NRCHUNK)
            def _():
                pltpu.async_copy(zrows, agg.at[pl.ds(j * 8, 8)], ssem)
            return 0
        lax.fori_loop(0, 8, fire, 0)

        def drain(i_, _2):
            j = (t * 8 + i_) * NS + s

            @pl.when(j < NRCHUNK)
            def _():
                pltpu.make_async_copy(zrows, agg.at[pl.ds(0, 8)],
                                      ssem).wait()
            return 0
        lax.fori_loop(0, 8, drain, 0)
        return 0

    wait_idx(0)
    pltpu.async_copy(y_hbm.at[srcb.at[0, 0]], rows.at[0], gsem)
    lax.fori_loop(0, -(-RITERS // 8), zero_batch, 0)
    plsc.subcore_barrier()


    def wait_scat(bb):
        pltpu.make_async_copy(rows.at[bb], agg.at[dstb.at[0, 0]],
                              ssem).wait()

    def sblock(i, _):
        for p in range(2):
            t = i * 2 + p
            for q in range(BROWS):
                bb = q % 2
                rr = w * QPT + t * BROWS + q

                @pl.when(rr < REALROWS)
                def _():
                    pltpu.make_async_copy(y_hbm.at[srcb.at[p, 0]],
                                          rows.at[bb], gsem).wait()
                if q == 0:
                    @pl.when((t > 0) & (rr < REALROWS))
                    def _():
                        wait_scat(1 - bb)
                else:
                    @pl.when(rr < REALROWS)
                    def _():
                        wait_scat(1 - bb)

                @pl.when(rr < REALROWS)
                def _():
                    pltpu.async_copy(rows.at[bb], agg.at[dstb.at[p, q]],
                                     ssem, add=True)
                if q < BROWS - 1:
                    @pl.when(rr + 1 < REALROWS)
                    def _():
                        pltpu.async_copy(y_hbm.at[srcb.at[p, q + 1]],
                                         rows.at[1 - bb], gsem)
                else:
                    @pl.when(t < NBLK - 1)
                    def _():
                        wait_idx(1 - p)

                    @pl.when((t < NBLK - 1) & (rr + 1 < REALROWS))
                    def _():
                        pltpu.async_copy(y_hbm.at[srcb.at[1 - p, 0]],
                                         rows.at[1 - bb], gsem)

            @pl.when(t + 2 < NBLK)
            def _():
                fire_idx(t + 2, p)
        return 0
    lax.fori_loop(0, NBLK // 2, sblock, 0)
    wait_scat(0)
    plsc.subcore_barrier()

    def out_batch(t, _):
        def fire(i_, _2):
            j = (t * 8 + i_) * NS + s

            @pl.when(j < NRCHUNK)
            def _():
                pltpu.async_copy(agg.at[pl.ds(j * 8, 8)],
                                 out_hbm.at[c, pl.ds(j * 8, 8)], gsem)
            return 0
        lax.fori_loop(0, 8, fire, 0)

        def drain(i_, _2):
            j = (t * 8 + i_) * NS + s

            @pl.when(j < NRCHUNK)
            def _():
                pltpu.make_async_copy(agg.at[pl.ds(0, 8)],
                                      out_hbm.at[c, pl.ds(0, 8)],
                                      gsem).wait()
            return 0
        lax.fori_loop(0, 8, drain, 0)
        return 0
    lax.fori_loop(0, -(-RITERS // 8), out_batch, 0)


_RB = 2048


def _prescale_matmul_body(deg_ref, x_ref, w_ref, y_ref):
    d = deg_ref[0, 0, :] + deg_ref[1, 0, :]
    ns = lax.rsqrt(jnp.maximum(d, 1.0))
    y_ref[...] = jnp.dot(x_ref[...] * ns[:, None], w_ref[...],
                         preferred_element_type=jnp.float32)


def _finish_body(deg_ref, b_ref, p_ref, o_ref):
    d = deg_ref[0, 1, :] + deg_ref[1, 1, :]
    nd = lax.rsqrt(jnp.maximum(d, 1.0))
    o_ref[...] = (p_ref[0] + p_ref[1]) * nd[:, None] + b_ref[...]


def kernel(features, edge_index, W, b):
    edge_index = edge_index.astype(jnp.int32)
    pad = jnp.full((EPAD - N_EDGES,), N_NODES, jnp.int32)
    src = jnp.concatenate([edge_index[0], pad]).reshape(IDXROWS, SUB)
    dst = jnp.concatenate([edge_index[1], pad]).reshape(IDXROWS, SUB)

    deg = _degree_kernel(src, dst)

    y = pl.pallas_call(
        _prescale_matmul_body,
        grid=(NPAD // _RB,),
        in_specs=[
            pl.BlockSpec((NC, 2, _RB), lambda i: (0, 0, i)),
            pl.BlockSpec((_RB, D), lambda i: (i, 0)),
            pl.BlockSpec((D, D), lambda i: (0, 0)),
        ],
        out_specs=pl.BlockSpec((_RB, D), lambda i: (i, 0)),
        out_shape=jax.ShapeDtypeStruct((NPAD, D), jnp.float32),
    )(deg, features, W)

    parts = _agg_kernel(y, src, dst)

    out = pl.pallas_call(
        _finish_body,
        grid=(pl.cdiv(N_NODES, _RB),),
        in_specs=[
            pl.BlockSpec((NC, 2, _RB), lambda i: (0, 0, i)),
            pl.BlockSpec((1, D), lambda i: (0, 0)),
            pl.BlockSpec((NC, _RB, D), lambda i: (0, i, 0)),
        ],
        out_specs=pl.BlockSpec((_RB, D), lambda i: (i, 0)),
        out_shape=jax.ShapeDtypeStruct((N_NODES, D), jnp.float32),
    )(deg, b.reshape(1, D), parts)

    return out
